# Initial kernel scaffold; baseline (speedup 1.0000x reference)
import jax
import jax.numpy as jnp
from jax.experimental import pallas as pl


def kernel(data, attract, repel, epochs):
    raise NotImplementedError("write your pallas kernel here")


# speedup vs baseline: 4.1664x; 4.1664x over previous
import jax
import jax.numpy as jnp
from jax.experimental import pallas as pl
from jax.experimental.pallas import tpu as pltpu

_A_O = 0.1
_K_O = 0.1
_A_ON = 0.006
_K_ON = 0.006
_A_R = 0.1
_K_R = 0.1
_A_RN = 0.006
_K_RN = 0.006
_MAXN = 16

_BIG_I = 2 ** 30
_INF = float("inf")


def _rayleigh(dx, k, a):
    return k / a * dx * jnp.exp(-0.5 * (dx / a) ** 2)


def _negexp(dx, k, a):
    return k * jnp.exp(-a * dx ** 2)


def _argmin_scalar(score):
    m = jnp.min(score)
    lane = jax.lax.broadcasted_iota(jnp.int32, score.shape, 1)
    idx = jnp.min(jnp.where(score == m, lane, _BIG_I))
    return m, idx


def _topk_mask_rows(score, k):
    lane = jax.lax.broadcasted_iota(jnp.int32, score.shape, 1)
    sel = jnp.zeros(score.shape, jnp.bool_)
    work = score
    for _ in range(k):
        m = jnp.min(work, axis=1, keepdims=True)
        rowidx = jnp.min(jnp.where(work == m, lane, _BIG_I), axis=1,
                         keepdims=True)
        hit = lane == rowidx
        sel = jnp.logical_or(sel, hit)
        work = jnp.where(hit, _INF, work)
    return sel


def _sum_k_smallest_rows(score, k):
    lane = jax.lax.broadcasted_iota(jnp.int32, score.shape, 1)
    work = score
    acc = jnp.zeros((score.shape[0], 1), jnp.float32)
    for _ in range(k):
        m = jnp.min(work, axis=1, keepdims=True)
        rowidx = jnp.min(jnp.where(work == m, lane, _BIG_I), axis=1,
                         keepdims=True)
        work = jnp.where(lane == rowidx, _INF, work)
        acc = acc + m
    return jnp.sum(acc)


def _ng_kernel(data_ref, dt_ref, attract_ref, at_ref, repel_ref, rt_ref,
               epochs_ref, best_ref, v_ref, adj_ref):
    n, d = data_ref.shape
    na = attract_ref.shape[0]
    nr = repel_ref.shape[0]

    v_ref[...] = data_ref[...]
    best_ref[...] = data_ref[...]
    adj_ref[...] = jnp.zeros((n, n), jnp.float32)

    sq_d = jnp.sum(dt_ref[...] ** 2, axis=0, keepdims=True)
    sq_a = jnp.sum(at_ref[...] ** 2, axis=0, keepdims=True)
    sq_r = jnp.sum(rt_ref[...] ** 2, axis=0, keepdims=True)

    lane_n = jax.lax.broadcasted_iota(jnp.int32, (1, n), 1)

    def node_step(s1, carry):
        v1 = v_ref[pl.ds(s1, 1), :]
        sq_v1 = jnp.sum(v1 * v1)

        ds_sc = sq_d - 2.0 * jnp.dot(v1, dt_ref[...],
                                     preferred_element_type=jnp.float32)
        work = ds_sc
        nbr = []
        for j in range(6):
            _, idx = _argmin_scalar(work)
            work = jnp.where(lane_n == idx, _INF, work)
            if j > 0:
                nbr.append(idx)

        row = adj_ref[pl.ds(s1, 1), :]
        mask5 = (lane_n == nbr[0])
        for idx in nbr[1:]:
            mask5 = jnp.logical_or(mask5, lane_n == idx)
        adj_ref[pl.ds(s1, 1), :] = jnp.where(mask5, 1.0, row)
        for idx in nbr:
            r2 = adj_ref[pl.ds(idx, 1), :]
            adj_ref[pl.ds(idx, 1), :] = jnp.where(lane_n == s1, 1.0, r2)

        da = sq_a - 2.0 * jnp.dot(v1, at_ref[...],
                                  preferred_element_type=jnp.float32)
        lane_a = jax.lax.broadcasted_iota(jnp.int32, (1, na), 1)
        work = da
        gks = []
        for j in range(3):
            _, idx = _argmin_scalar(work)
            work = jnp.where(lane_a == idx, _INF, work)
            gks.append(attract_ref[pl.ds(idx, 1), :])

        move = jnp.zeros((1, d), jnp.float32)
        for g in gks:
            diff = g - v1
            dx = jnp.sum(diff * diff)
            move = move + _rayleigh(dx, _A_O, _K_O) * diff

        dxr = sq_r + sq_v1 - 2.0 * jnp.dot(v1, rt_ref[...],
                                           preferred_element_type=jnp.float32)
        sel_r = _topk_mask_rows(dxr, 10)
        w_r = jnp.where(sel_r, _negexp(dxr, _A_R, _K_R), 0.0)
        move = move - (jnp.dot(w_r, repel_ref[...],
                               preferred_element_type=jnp.float32)
                       - jnp.sum(w_r) * v1)
        v_ref[pl.ds(s1, 1), :] = v1 + move

        workn = adj_ref[pl.ds(s1, 1), :]
        nidx = []
        valid = []
        for j in range(_MAXN):
            m = jnp.max(workn)
            idx = jnp.min(jnp.where(workn == m, lane_n, _BIG_I))
            workn = jnp.where(lane_n == idx, -1.0, workn)
            nidx.append(idx)
            valid.append(m > 0.0)

        vk = jnp.concatenate([v_ref[pl.ds(i, 1), :] for i in nidx], axis=0)
        sq_vk = jnp.sum(vk * vk, axis=1, keepdims=True)

        mk = jnp.zeros((_MAXN, d), jnp.float32)
        for g in gks:
            diffn = g - vk
            dxn = jnp.sum(diffn * diffn, axis=1, keepdims=True)
            mk = mk + _rayleigh(dxn, _A_ON, _K_ON) * diffn

        drn = sq_vk + sq_r - 2.0 * jnp.dot(vk, rt_ref[...],
                                           preferred_element_type=jnp.float32)
        sel_n = _topk_mask_rows(drn, 10)
        w_n = jnp.where(sel_n, _negexp(drn, _A_RN, _K_RN), 0.0)
        w_sum = jnp.sum(w_n, axis=1, keepdims=True)
        mk = mk - (jnp.dot(w_n, repel_ref[...],
                           preferred_element_type=jnp.float32) - w_sum * vk)

        for j in range(_MAXN):
            row_new = jnp.where(valid[j], vk[j:j + 1, :] + mk[j:j + 1, :],
                                vk[j:j + 1, :])
            v_ref[pl.ds(nidx[j], 1), :] = row_new
        return carry

    def epoch_body(e, prev):
        jax.lax.fori_loop(0, n, node_step, 0)
        vv = v_ref[...]
        sq_v = jnp.sum(vv * vv, axis=1, keepdims=True)
        dall = sq_v + sq_a - 2.0 * jnp.dot(vv, at_ref[...],
                                           preferred_element_type=jnp.float32)
        cur = _sum_k_smallest_rows(dall, 20)
        improved = jnp.logical_not(prev < cur)
        best_ref[...] = jnp.where(improved, v_ref[...], best_ref[...])
        return jnp.where(improved, cur, prev)

    jax.lax.fori_loop(0, epochs_ref[0], epoch_body, _INF)


def kernel(data, attract, repel, epochs):
    n, d = data.shape
    epochs_arr = jnp.asarray(epochs, jnp.int32).reshape(1)
    return pl.pallas_call(
        _ng_kernel,
        out_shape=jax.ShapeDtypeStruct((n, d), jnp.float32),
        in_specs=[
            pl.BlockSpec(memory_space=pltpu.VMEM),
            pl.BlockSpec(memory_space=pltpu.VMEM),
            pl.BlockSpec(memory_space=pltpu.VMEM),
            pl.BlockSpec(memory_space=pltpu.VMEM),
            pl.BlockSpec(memory_space=pltpu.VMEM),
            pl.BlockSpec(memory_space=pltpu.VMEM),
            pl.BlockSpec(memory_space=pltpu.SMEM),
        ],
        out_specs=pl.BlockSpec(memory_space=pltpu.VMEM),
        scratch_shapes=[
            pltpu.VMEM((n, d), jnp.float32),
            pltpu.VMEM((n, n), jnp.float32),
        ],
    )(data, data.T, attract, attract.T, repel, repel.T, epochs_arr)


# speedup vs baseline: 7.1792x; 1.7231x over previous
import jax
import jax.numpy as jnp
from jax.experimental import pallas as pl
from jax.experimental.pallas import tpu as pltpu

_A_O = 0.1
_K_O = 0.1
_A_ON = 0.006
_K_ON = 0.006
_A_R = 0.1
_K_R = 0.1
_A_RN = 0.006
_K_RN = 0.006
_MAXN = 16

_BIG_I = 2 ** 30
_INF = float("inf")


def _rayleigh(dx, k, a):
    return k / a * dx * jnp.exp(-0.5 * (dx / a) ** 2)


def _negexp(dx, k, a):
    return k * jnp.exp(-a * dx ** 2)


def _topk_mask_rows(score, k):
    sel = jnp.zeros(score.shape, jnp.bool_)
    work = score
    for _ in range(k):
        m = jnp.min(work, axis=1, keepdims=True)
        hit = work == m
        sel = jnp.logical_or(sel, hit)
        work = jnp.where(hit, _INF, work)
    return sel


def _sum_k_smallest_rows(score, k):
    work = score
    acc = jnp.zeros((score.shape[0], 1), jnp.float32)
    for _ in range(k):
        m = jnp.min(work, axis=1, keepdims=True)
        work = jnp.where(work == m, _INF, work)
        acc = acc + m
    return jnp.sum(acc)


def _ng_kernel(data_ref, dt_ref, attract_ref, at_ref, repel_ref, rt_ref,
               epochs_ref, best_ref, v_ref, adj_ref):
    n, d = data_ref.shape
    na = attract_ref.shape[0]
    nr = repel_ref.shape[0]

    v_ref[...] = data_ref[...]
    best_ref[...] = data_ref[...]
    adj_ref[...] = jnp.zeros((n, n), jnp.float32)

    sq_d = jnp.sum(dt_ref[...] ** 2, axis=0, keepdims=True)
    sq_a = jnp.sum(at_ref[...] ** 2, axis=0, keepdims=True)
    sq_r = jnp.sum(rt_ref[...] ** 2, axis=0, keepdims=True)

    lane_n = jax.lax.broadcasted_iota(jnp.int32, (1, n), 1)

    def node_step(s1, carry):
        v1 = v_ref[pl.ds(s1, 1), :]
        sq_v1 = jnp.sum(v1 * v1)

        ds_sc = sq_d - 2.0 * jnp.dot(v1, dt_ref[...],
                                     preferred_element_type=jnp.float32)
        work = ds_sc
        nbr = []
        for j in range(6):
            m = jnp.min(work)
            hit = work == m
            if j > 0:
                nbr.append(jnp.min(jnp.where(hit, lane_n, _BIG_I)))
            work = jnp.where(hit, _INF, work)

        row = adj_ref[pl.ds(s1, 1), :]
        mask5 = (lane_n == nbr[0])
        for idx in nbr[1:]:
            mask5 = jnp.logical_or(mask5, lane_n == idx)
        adj_ref[pl.ds(s1, 1), :] = jnp.where(mask5, 1.0, row)
        for idx in nbr:
            r2 = adj_ref[pl.ds(idx, 1), :]
            adj_ref[pl.ds(idx, 1), :] = jnp.where(lane_n == s1, 1.0, r2)

        da = sq_a - 2.0 * jnp.dot(v1, at_ref[...],
                                  preferred_element_type=jnp.float32)
        lane_a = jax.lax.broadcasted_iota(jnp.int32, (1, na), 1)
        work = da
        gks = []
        for j in range(3):
            m = jnp.min(work)
            hit = work == m
            idx = jnp.min(jnp.where(hit, lane_a, _BIG_I))
            work = jnp.where(hit, _INF, work)
            gks.append(attract_ref[pl.ds(idx, 1), :])

        move = jnp.zeros((1, d), jnp.float32)
        for g in gks:
            diff = g - v1
            dx = jnp.sum(diff * diff)
            move = move + _rayleigh(dx, _A_O, _K_O) * diff

        dxr = sq_r + sq_v1 - 2.0 * jnp.dot(v1, rt_ref[...],
                                           preferred_element_type=jnp.float32)
        sel_r = _topk_mask_rows(dxr, 10)
        w_r = jnp.where(sel_r, _negexp(dxr, _A_R, _K_R), 0.0)
        move = move - (jnp.dot(w_r, repel_ref[...],
                               preferred_element_type=jnp.float32)
                       - jnp.sum(w_r) * v1)
        v_ref[pl.ds(s1, 1), :] = v1 + move

        rowv = adj_ref[pl.ds(s1, 1), :]
        keys = jnp.where(rowv > 0.0, lane_n, lane_n + n)
        nidx = []
        valid = []
        workn = keys
        for j in range(_MAXN):
            m = jnp.min(workn)
            workn = jnp.where(workn == m, _BIG_I, workn)
            valid.append(m < n)
            nidx.append(jnp.where(m < n, m, m - n))

        vk = jnp.concatenate([v_ref[pl.ds(i, 1), :] for i in nidx], axis=0)
        sq_vk = jnp.sum(vk * vk, axis=1, keepdims=True)

        mk = jnp.zeros((_MAXN, d), jnp.float32)
        for g in gks:
            diffn = g - vk
            dxn = jnp.sum(diffn * diffn, axis=1, keepdims=True)
            mk = mk + _rayleigh(dxn, _A_ON, _K_ON) * diffn

        drn = sq_vk + sq_r - 2.0 * jnp.dot(vk, rt_ref[...],
                                           preferred_element_type=jnp.float32)
        sel_n = _topk_mask_rows(drn, 10)
        w_n = jnp.where(sel_n, _negexp(drn, _A_RN, _K_RN), 0.0)
        w_sum = jnp.sum(w_n, axis=1, keepdims=True)
        mk = mk - (jnp.dot(w_n, repel_ref[...],
                           preferred_element_type=jnp.float32) - w_sum * vk)

        for j in range(_MAXN):
            row_new = jnp.where(valid[j], vk[j:j + 1, :] + mk[j:j + 1, :],
                                vk[j:j + 1, :])
            v_ref[pl.ds(nidx[j], 1), :] = row_new
        return carry

    def epoch_body(e, prev):
        jax.lax.fori_loop(0, n, node_step, 0)
        vv = v_ref[...]
        sq_v = jnp.sum(vv * vv, axis=1, keepdims=True)
        dall = sq_v + sq_a - 2.0 * jnp.dot(vv, at_ref[...],
                                           preferred_element_type=jnp.float32)
        cur = _sum_k_smallest_rows(dall, 20)
        improved = jnp.logical_not(prev < cur)
        best_ref[...] = jnp.where(improved, v_ref[...], best_ref[...])
        return jnp.where(improved, cur, prev)

    jax.lax.fori_loop(0, epochs_ref[0], epoch_body, _INF)


def kernel(data, attract, repel, epochs):
    n, d = data.shape
    epochs_arr = jnp.asarray(epochs, jnp.int32).reshape(1)
    return pl.pallas_call(
        _ng_kernel,
        out_shape=jax.ShapeDtypeStruct((n, d), jnp.float32),
        in_specs=[
            pl.BlockSpec(memory_space=pltpu.VMEM),
            pl.BlockSpec(memory_space=pltpu.VMEM),
            pl.BlockSpec(memory_space=pltpu.VMEM),
            pl.BlockSpec(memory_space=pltpu.VMEM),
            pl.BlockSpec(memory_space=pltpu.VMEM),
            pl.BlockSpec(memory_space=pltpu.VMEM),
            pl.BlockSpec(memory_space=pltpu.SMEM),
        ],
        out_specs=pl.BlockSpec(memory_space=pltpu.VMEM),
        scratch_shapes=[
            pltpu.VMEM((n, d), jnp.float32),
            pltpu.VMEM((n, n), jnp.float32),
        ],
    )(data, data.T, attract, attract.T, repel, repel.T, epochs_arr)
